# baseline (speedup 1.0000x reference)
import functools
import math

import jax
import jax.numpy as jnp
from jax import lax
from jax.experimental import pallas as pl
from jax.experimental.pallas import tpu as pltpu
from jax.experimental.pallas import tpu_sc as plsc

N = 10000
E = 320000
D = 128
H = 8
C = 16

NC = 2
NS = 16
L = 16

CHUNK = 80
TILES = NC * NS
EDGES_PER_TILE = E // TILES
NUM_CHUNKS = EDGES_PER_TILE // CHUNK
GROUPS = CHUNK // L
ROWW = 136
ROWS_PER_TILE = N // NS

_INV_SQRT_C = 1.0 / math.sqrt(C)


def _proj_body(x_ref, wq_ref, bq_ref, wk_ref, bk_ref, wv_ref, bv_ref,
               ws_ref, bs_ref, q_ref, kv_ref, skip_ref):
    x = x_ref[...]
    dn = (((1,), (1,)), ((), ()))
    f32 = jnp.float32
    q_ref[...] = lax.dot_general(x, wq_ref[...], dn, preferred_element_type=f32) + bq_ref[...]
    k = lax.dot_general(x, wk_ref[...], dn, preferred_element_type=f32) + bk_ref[...]
    v = lax.dot_general(x, wv_ref[...], dn, preferred_element_type=f32) + bv_ref[...]
    kv_ref[...] = jnp.concatenate([k, v], axis=1)
    skip_ref[...] = x + lax.dot_general(x, ws_ref[...], dn, preferred_element_type=f32) + bs_ref[...]


def _project(x, Wq, bq, Wk, bk, Wv, bv, Wskip, bskip):
    blk = 2000
    grid = (N // blk,)
    full128 = pl.BlockSpec((D, D), lambda i: (0, 0))
    bias = pl.BlockSpec((1, D), lambda i: (0, 0))
    return pl.pallas_call(
        _proj_body,
        grid=grid,
        in_specs=[
            pl.BlockSpec((blk, D), lambda i: (i, 0)),
            full128, bias, full128, bias, full128, bias, full128, bias,
        ],
        out_specs=[
            pl.BlockSpec((blk, D), lambda i: (i, 0)),
            pl.BlockSpec((blk, 2 * D), lambda i: (i, 0)),
            pl.BlockSpec((blk, D), lambda i: (i, 0)),
        ],
        out_shape=[
            jax.ShapeDtypeStruct((N, D), jnp.float32),
            jax.ShapeDtypeStruct((N, 2 * D), jnp.float32),
            jax.ShapeDtypeStruct((N, D), jnp.float32),
        ],
    )(x, Wq, bq.reshape(1, D), Wk, bk.reshape(1, D),
      Wv, bv.reshape(1, D), Wskip, bskip.reshape(1, D))


def _edge_body(q_hbm, kv_hbm, src_hbm, dst_hbm, z_hbm, out_hbm,
               sb0, db0, qd0, kv0, msg0, acc_sh, sem_g0):
    c = lax.axis_index("c")
    s = lax.axis_index("s")

    iota16 = lax.iota(jnp.int32, L)

    base = s * ROWS_PER_TILE
    pltpu.sync_copy(z_hbm, acc_sh.at[pl.ds(base, ROWS_PER_TILE)])
    plsc.subcore_barrier()

    row0 = (c * NS + s) * NUM_CHUNKS

    lane_vecs = [jnp.full((L,), lane, jnp.int32) for lane in range(L)]
    hcols = [h * C + iota16 for h in range(H)]
    last_lane = jnp.full((L,), L - 1, jnp.int32)
    lane0_mask = iota16 == 0
    gdn = lax.GatherDimensionNumbers(offset_dims=(),
                                     collapsed_slice_dims=(0,),
                                     start_index_map=(0,))

    def compute(qd, kvb, msg):
        @plsc.parallel_loop(0, CHUNK, unroll=8)
        def _edge_loop(e):
            ecol = lane_vecs[0] + e
            for h in range(H):
                qh = plsc.load_gather(qd, [ecol, hcols[h]])
                kh = plsc.load_gather(kvb, [ecol, hcols[h]])
                t = plsc.cumsum(qh * kh)
                tot = lax.gather(t, last_lane[:, None], gdn, (1,),
                                 mode=lax.GatherScatterMode.PROMISE_IN_BOUNDS)
                w = jnp.exp(tot * _INV_SQRT_C)
                vh = plsc.load_gather(kvb, [ecol, D + hcols[h]])
                plsc.store_scatter(msg, [ecol, hcols[h]], w * vh)
                plsc.store_scatter(msg, [ecol, lane_vecs[0] + (D + h)],
                                   w, mask=lane0_mask)

    def chunk_body(i, _):
        pltpu.sync_copy(src_hbm.at[row0 + i], sb0)
        pltpu.sync_copy(dst_hbm.at[row0 + i], db0)
        cq = pltpu.async_copy(q_hbm.at[db0], qd0, sem_g0)
        ck = pltpu.async_copy(kv_hbm.at[sb0], kv0, sem_g0)
        cq.wait()
        ck.wait()
        compute(qd0, kv0, msg0)
        pltpu.sync_copy(msg0, acc_sh.at[db0], add=True)
        return 0

    lax.fori_loop(0, NUM_CHUNKS, chunk_body, 0)

    plsc.subcore_barrier()
    out_base = c * N + base
    pltpu.sync_copy(acc_sh.at[pl.ds(base, ROWS_PER_TILE)],
                    out_hbm.at[pl.ds(out_base, ROWS_PER_TILE)])


def _edge_sc(q, kv, src, dst, zeros):
    mesh = plsc.VectorSubcoreMesh(core_axis_name="c", subcore_axis_name="s",
                                  num_cores=NC, num_subcores=NS)
    f = pl.kernel(
        _edge_body,
        out_type=jax.ShapeDtypeStruct((NC * N, ROWW), jnp.float32),
        mesh=mesh,
        scratch_types=[
            pltpu.VMEM((CHUNK,), jnp.int32),
            pltpu.VMEM((CHUNK,), jnp.int32),
            pltpu.VMEM((CHUNK, D), jnp.float32),
            pltpu.VMEM((CHUNK, 2 * D), jnp.float32),
            pltpu.VMEM((CHUNK, ROWW), jnp.float32),
            pltpu.VMEM_SHARED((N, ROWW), jnp.float32),
            pltpu.SemaphoreType.DMA,
        ],
        compiler_params=pltpu.CompilerParams(use_tc_tiling_on_sc=False,
                                             needs_layout_passes=False),
    )
    return f(q, kv, src, dst, zeros)


def _combine_body(acc_ref, skip_ref, p_ref, out_ref):
    a0 = acc_ref[0]
    a1 = acc_ref[1]
    num = a0[:, :D] + a1[:, :D]
    den = a0[:, D:D + H] + a1[:, D:D + H]
    r = 1.0 / (den + 1e-16)
    rexp = lax.dot_general(r, p_ref[...], (((1,), (0,)), ((), ())),
                           preferred_element_type=jnp.float32)
    out_ref[...] = skip_ref[...] + num * rexp


def _combine(acc, skipx, P):
    blk = 2000
    grid = (N // blk,)
    return pl.pallas_call(
        _combine_body,
        grid=grid,
        in_specs=[
            pl.BlockSpec((NC, blk, ROWW), lambda i: (0, i, 0)),
            pl.BlockSpec((blk, D), lambda i: (i, 0)),
            pl.BlockSpec((H, D), lambda i: (0, 0)),
        ],
        out_specs=pl.BlockSpec((blk, D), lambda i: (i, 0)),
        out_shape=jax.ShapeDtypeStruct((N, D), jnp.float32),
    )(acc, skipx, P)


@jax.jit
def kernel(x, edge_index, Wq, bq, Wk, bk, Wv, bv, Wskip, bskip):
    ei = edge_index.astype(jnp.int32)
    src = ei[0].reshape(E // CHUNK, CHUNK)
    dst = ei[1].reshape(E // CHUNK, CHUNK)
    q, kv, skipx = _project(x, Wq, bq, Wk, bk, Wv, bv, Wskip, bskip)
    zeros = jnp.zeros((ROWS_PER_TILE, ROWW), jnp.float32)
    acc_flat = _edge_sc(q, kv, src, dst, zeros)
    acc = acc_flat.reshape(NC, N, ROWW)
    P = jnp.repeat(jnp.eye(H, dtype=jnp.float32), C, axis=1)
    return _combine(acc, skipx, P)


# speedup vs baseline: 2.7774x; 2.7774x over previous
import functools
import math

import jax
import jax.numpy as jnp
from jax import lax
from jax.experimental import pallas as pl
from jax.experimental.pallas import tpu as pltpu
from jax.experimental.pallas import tpu_sc as plsc

N = 10000
E = 320000
D = 128
H = 8
C = 16

NC = 2
NS = 16
L = 16

CHUNK = 80
TILES = NC * NS
EDGES_PER_TILE = E // TILES
NUM_CHUNKS = EDGES_PER_TILE // CHUNK
GROUPS = CHUNK // L
ROWW = 136
ROWS_PER_TILE = N // NS

_INV_SQRT_C = 1.0 / math.sqrt(C)


def _proj_body(x_ref, wq_ref, bq_ref, wk_ref, bk_ref, wv_ref, bv_ref,
               ws_ref, bs_ref, q_ref, kv_ref, skip_ref):
    x = x_ref[...]
    dn = (((1,), (1,)), ((), ()))
    f32 = jnp.float32
    q_ref[...] = lax.dot_general(x, wq_ref[...], dn, preferred_element_type=f32) + bq_ref[...]
    k = lax.dot_general(x, wk_ref[...], dn, preferred_element_type=f32) + bk_ref[...]
    v = lax.dot_general(x, wv_ref[...], dn, preferred_element_type=f32) + bv_ref[...]
    kv_ref[...] = jnp.concatenate([k, v], axis=1)
    skip_ref[...] = x + lax.dot_general(x, ws_ref[...], dn, preferred_element_type=f32) + bs_ref[...]


def _project(x, Wq, bq, Wk, bk, Wv, bv, Wskip, bskip):
    blk = 2000
    grid = (N // blk,)
    full128 = pl.BlockSpec((D, D), lambda i: (0, 0))
    bias = pl.BlockSpec((1, D), lambda i: (0, 0))
    return pl.pallas_call(
        _proj_body,
        grid=grid,
        in_specs=[
            pl.BlockSpec((blk, D), lambda i: (i, 0)),
            full128, bias, full128, bias, full128, bias, full128, bias,
        ],
        out_specs=[
            pl.BlockSpec((blk, D), lambda i: (i, 0)),
            pl.BlockSpec((blk, 2 * D), lambda i: (i, 0)),
            pl.BlockSpec((blk, D), lambda i: (i, 0)),
        ],
        out_shape=[
            jax.ShapeDtypeStruct((N, D), jnp.float32),
            jax.ShapeDtypeStruct((N, 2 * D), jnp.float32),
            jax.ShapeDtypeStruct((N, D), jnp.float32),
        ],
    )(x, Wq, bq.reshape(1, D), Wk, bk.reshape(1, D),
      Wv, bv.reshape(1, D), Wskip, bskip.reshape(1, D))


def _edge_body(q_hbm, kv_hbm, src_hbm, dst_hbm, z_hbm, out_hbm,
               sb0, db0, qd0, kv0, msg0, acc_sh, sem_g0):
    c = lax.axis_index("c")
    s = lax.axis_index("s")

    iota16 = lax.iota(jnp.int32, L)

    base = s * ROWS_PER_TILE
    pltpu.sync_copy(z_hbm, acc_sh.at[pl.ds(base, ROWS_PER_TILE)])
    plsc.subcore_barrier()

    row0 = (c * NS + s) * NUM_CHUNKS

    lane_vecs = [jnp.full((L,), lane, jnp.int32) for lane in range(L)]
    hcols = [h * C + iota16 for h in range(H)]
    last_lane = jnp.full((L,), L - 1, jnp.int32)
    lane0_mask = iota16 == 0
    gdn = lax.GatherDimensionNumbers(offset_dims=(),
                                     collapsed_slice_dims=(0,),
                                     start_index_map=(0,))

    def compute(qd, kvb, msg):
        @plsc.parallel_loop(0, CHUNK, unroll=2)
        def _edge_loop(e):
            ecol = lane_vecs[0] + e
            for h in range(H):
                qh = plsc.load_gather(qd, [ecol, hcols[h]])
                kh = plsc.load_gather(kvb, [ecol, hcols[h]])
                t = plsc.cumsum(qh * kh)
                tot = lax.gather(t, last_lane[:, None], gdn, (1,),
                                 mode=lax.GatherScatterMode.PROMISE_IN_BOUNDS)
                w = jnp.exp(tot * _INV_SQRT_C)
                vh = plsc.load_gather(kvb, [ecol, D + hcols[h]])
                plsc.store_scatter(msg, [ecol, hcols[h]], w * vh)
                plsc.store_scatter(msg, [ecol, lane_vecs[0] + (D + h)],
                                   w, mask=lane0_mask)

    def chunk_body(i, _):
        pltpu.sync_copy(src_hbm.at[row0 + i], sb0)
        pltpu.sync_copy(dst_hbm.at[row0 + i], db0)
        cq = pltpu.async_copy(q_hbm.at[db0], qd0, sem_g0)
        ck = pltpu.async_copy(kv_hbm.at[sb0], kv0, sem_g0)
        cq.wait()
        ck.wait()
        compute(qd0, kv0, msg0)
        pltpu.sync_copy(msg0, acc_sh.at[db0], add=True)
        return 0

    lax.fori_loop(0, NUM_CHUNKS, chunk_body, 0)

    plsc.subcore_barrier()
    out_base = c * N + base
    pltpu.sync_copy(acc_sh.at[pl.ds(base, ROWS_PER_TILE)],
                    out_hbm.at[pl.ds(out_base, ROWS_PER_TILE)])


def _edge_sc(q, kv, src, dst, zeros):
    mesh = plsc.VectorSubcoreMesh(core_axis_name="c", subcore_axis_name="s",
                                  num_cores=NC, num_subcores=NS)
    f = pl.kernel(
        _edge_body,
        out_type=jax.ShapeDtypeStruct((NC * N, ROWW), jnp.float32),
        mesh=mesh,
        scratch_types=[
            pltpu.VMEM((CHUNK,), jnp.int32),
            pltpu.VMEM((CHUNK,), jnp.int32),
            pltpu.VMEM((CHUNK, D), jnp.float32),
            pltpu.VMEM((CHUNK, 2 * D), jnp.float32),
            pltpu.VMEM((CHUNK, ROWW), jnp.float32),
            pltpu.VMEM_SHARED((N, ROWW), jnp.float32),
            pltpu.SemaphoreType.DMA,
        ],
        compiler_params=pltpu.CompilerParams(use_tc_tiling_on_sc=False,
                                             needs_layout_passes=False),
    )
    return f(q, kv, src, dst, zeros)


def _combine_body(acc_ref, skip_ref, p_ref, out_ref):
    a0 = acc_ref[0]
    a1 = acc_ref[1]
    num = a0[:, :D] + a1[:, :D]
    den = a0[:, D:D + H] + a1[:, D:D + H]
    r = 1.0 / (den + 1e-16)
    rexp = lax.dot_general(r, p_ref[...], (((1,), (0,)), ((), ())),
                           preferred_element_type=jnp.float32)
    out_ref[...] = skip_ref[...] + num * rexp


def _combine(acc, skipx, P):
    blk = 2000
    grid = (N // blk,)
    return pl.pallas_call(
        _combine_body,
        grid=grid,
        in_specs=[
            pl.BlockSpec((NC, blk, ROWW), lambda i: (0, i, 0)),
            pl.BlockSpec((blk, D), lambda i: (i, 0)),
            pl.BlockSpec((H, D), lambda i: (0, 0)),
        ],
        out_specs=pl.BlockSpec((blk, D), lambda i: (i, 0)),
        out_shape=jax.ShapeDtypeStruct((N, D), jnp.float32),
    )(acc, skipx, P)


@jax.jit
def kernel(x, edge_index, Wq, bq, Wk, bk, Wv, bv, Wskip, bskip):
    ei = edge_index.astype(jnp.int32)
    src = ei[0].reshape(E // CHUNK, CHUNK)
    dst = ei[1].reshape(E // CHUNK, CHUNK)
    q, kv, skipx = _project(x, Wq, bq, Wk, bk, Wv, bv, Wskip, bskip)
    zeros = jnp.zeros((ROWS_PER_TILE, ROWW), jnp.float32)
    acc_flat = _edge_sc(q, kv, src, dst, zeros)
    acc = acc_flat.reshape(NC, N, ROWW)
    P = jnp.repeat(jnp.eye(H, dtype=jnp.float32), C, axis=1)
    return _combine(acc, skipx, P)


# speedup vs baseline: 2.8580x; 1.0290x over previous
import functools
import math

import jax
import jax.numpy as jnp
from jax import lax
from jax.experimental import pallas as pl
from jax.experimental.pallas import tpu as pltpu
from jax.experimental.pallas import tpu_sc as plsc

N = 10000
E = 320000
D = 128
H = 8
C = 16

NC = 2
NS = 16
L = 16

CHUNK = 80
TILES = NC * NS
EDGES_PER_TILE = E // TILES
NUM_CHUNKS = EDGES_PER_TILE // CHUNK
GROUPS = CHUNK // L
ROWW = 136
ROWS_PER_TILE = N // NS

_INV_SQRT_C = 1.0 / math.sqrt(C)


def _proj_body(x_ref, wq_ref, bq_ref, wk_ref, bk_ref, wv_ref, bv_ref,
               ws_ref, bs_ref, q_ref, kv_ref, skip_ref):
    x = x_ref[...]
    dn = (((1,), (1,)), ((), ()))
    f32 = jnp.float32
    q_ref[...] = lax.dot_general(x, wq_ref[...], dn, preferred_element_type=f32) + bq_ref[...]
    k = lax.dot_general(x, wk_ref[...], dn, preferred_element_type=f32) + bk_ref[...]
    v = lax.dot_general(x, wv_ref[...], dn, preferred_element_type=f32) + bv_ref[...]
    kv_ref[...] = jnp.concatenate([k, v], axis=1)
    skip_ref[...] = x + lax.dot_general(x, ws_ref[...], dn, preferred_element_type=f32) + bs_ref[...]


def _project(x, Wq, bq, Wk, bk, Wv, bv, Wskip, bskip):
    blk = 2000
    grid = (N // blk,)
    full128 = pl.BlockSpec((D, D), lambda i: (0, 0))
    bias = pl.BlockSpec((1, D), lambda i: (0, 0))
    return pl.pallas_call(
        _proj_body,
        grid=grid,
        in_specs=[
            pl.BlockSpec((blk, D), lambda i: (i, 0)),
            full128, bias, full128, bias, full128, bias, full128, bias,
        ],
        out_specs=[
            pl.BlockSpec((blk, D), lambda i: (i, 0)),
            pl.BlockSpec((blk, 2 * D), lambda i: (i, 0)),
            pl.BlockSpec((blk, D), lambda i: (i, 0)),
        ],
        out_shape=[
            jax.ShapeDtypeStruct((N, D), jnp.float32),
            jax.ShapeDtypeStruct((N, 2 * D), jnp.float32),
            jax.ShapeDtypeStruct((N, D), jnp.float32),
        ],
    )(x, Wq, bq.reshape(1, D), Wk, bk.reshape(1, D),
      Wv, bv.reshape(1, D), Wskip, bskip.reshape(1, D))


def _edge_body(q_hbm, kv_hbm, src_hbm, dst_hbm, z_hbm, out_hbm,
               sb0, db0, qd0, kv0, msg0, acc_sh, sem_g0):
    c = lax.axis_index("c")
    s = lax.axis_index("s")

    iota16 = lax.iota(jnp.int32, L)

    base = s * ROWS_PER_TILE
    pltpu.sync_copy(z_hbm, acc_sh.at[pl.ds(base, ROWS_PER_TILE)])
    plsc.subcore_barrier()

    row0 = (c * NS + s) * NUM_CHUNKS

    lane_vecs = [jnp.full((L,), lane, jnp.int32) for lane in range(L)]
    hcols = [h * C + iota16 for h in range(H)]
    last_lane = jnp.full((L,), L - 1, jnp.int32)
    lane0_mask = iota16 == 0
    gdn = lax.GatherDimensionNumbers(offset_dims=(),
                                     collapsed_slice_dims=(0,),
                                     start_index_map=(0,))

    def compute(qd, kvb, msg):
        @plsc.parallel_loop(0, CHUNK, unroll=4)
        def _edge_loop(e):
            ecol = lane_vecs[0] + e
            for h in range(H):
                qh = plsc.load_gather(qd, [ecol, hcols[h]])
                kh = plsc.load_gather(kvb, [ecol, hcols[h]])
                t = plsc.cumsum(qh * kh)
                tot = lax.gather(t, last_lane[:, None], gdn, (1,),
                                 mode=lax.GatherScatterMode.PROMISE_IN_BOUNDS)
                w = jnp.exp(tot * _INV_SQRT_C)
                vh = plsc.load_gather(kvb, [ecol, D + hcols[h]])
                plsc.store_scatter(msg, [ecol, hcols[h]], w * vh)
                plsc.store_scatter(msg, [ecol, lane_vecs[0] + (D + h)],
                                   w, mask=lane0_mask)

    def chunk_body(i, _):
        pltpu.sync_copy(src_hbm.at[row0 + i], sb0)
        pltpu.sync_copy(dst_hbm.at[row0 + i], db0)
        cq = pltpu.async_copy(q_hbm.at[db0], qd0, sem_g0)
        ck = pltpu.async_copy(kv_hbm.at[sb0], kv0, sem_g0)
        cq.wait()
        ck.wait()
        compute(qd0, kv0, msg0)
        pltpu.sync_copy(msg0, acc_sh.at[db0], add=True)
        return 0

    lax.fori_loop(0, NUM_CHUNKS, chunk_body, 0)

    plsc.subcore_barrier()
    out_base = c * N + base
    pltpu.sync_copy(acc_sh.at[pl.ds(base, ROWS_PER_TILE)],
                    out_hbm.at[pl.ds(out_base, ROWS_PER_TILE)])


def _edge_sc(q, kv, src, dst, zeros):
    mesh = plsc.VectorSubcoreMesh(core_axis_name="c", subcore_axis_name="s",
                                  num_cores=NC, num_subcores=NS)
    f = pl.kernel(
        _edge_body,
        out_type=jax.ShapeDtypeStruct((NC * N, ROWW), jnp.float32),
        mesh=mesh,
        scratch_types=[
            pltpu.VMEM((CHUNK,), jnp.int32),
            pltpu.VMEM((CHUNK,), jnp.int32),
            pltpu.VMEM((CHUNK, D), jnp.float32),
            pltpu.VMEM((CHUNK, 2 * D), jnp.float32),
            pltpu.VMEM((CHUNK, ROWW), jnp.float32),
            pltpu.VMEM_SHARED((N, ROWW), jnp.float32),
            pltpu.SemaphoreType.DMA,
        ],
        compiler_params=pltpu.CompilerParams(use_tc_tiling_on_sc=False,
                                             needs_layout_passes=False),
    )
    return f(q, kv, src, dst, zeros)


def _combine_body(acc_ref, skip_ref, p_ref, out_ref):
    a0 = acc_ref[0]
    a1 = acc_ref[1]
    num = a0[:, :D] + a1[:, :D]
    den = a0[:, D:D + H] + a1[:, D:D + H]
    r = 1.0 / (den + 1e-16)
    rexp = lax.dot_general(r, p_ref[...], (((1,), (0,)), ((), ())),
                           preferred_element_type=jnp.float32)
    out_ref[...] = skip_ref[...] + num * rexp


def _combine(acc, skipx, P):
    blk = 2000
    grid = (N // blk,)
    return pl.pallas_call(
        _combine_body,
        grid=grid,
        in_specs=[
            pl.BlockSpec((NC, blk, ROWW), lambda i: (0, i, 0)),
            pl.BlockSpec((blk, D), lambda i: (i, 0)),
            pl.BlockSpec((H, D), lambda i: (0, 0)),
        ],
        out_specs=pl.BlockSpec((blk, D), lambda i: (i, 0)),
        out_shape=jax.ShapeDtypeStruct((N, D), jnp.float32),
    )(acc, skipx, P)


@jax.jit
def kernel(x, edge_index, Wq, bq, Wk, bk, Wv, bv, Wskip, bskip):
    ei = edge_index.astype(jnp.int32)
    src = ei[0].reshape(E // CHUNK, CHUNK)
    dst = ei[1].reshape(E // CHUNK, CHUNK)
    q, kv, skipx = _project(x, Wq, bq, Wk, bk, Wv, bv, Wskip, bskip)
    zeros = jnp.zeros((ROWS_PER_TILE, ROWW), jnp.float32)
    acc_flat = _edge_sc(q, kv, src, dst, zeros)
    acc = acc_flat.reshape(NC, N, ROWW)
    P = jnp.repeat(jnp.eye(H, dtype=jnp.float32), C, axis=1)
    return _combine(acc, skipx, P)


# speedup vs baseline: 2.8610x; 1.0011x over previous
import math

import jax
import jax.numpy as jnp
from jax import lax
from jax.experimental import pallas as pl
from jax.experimental.pallas import tpu as pltpu
from jax.experimental.pallas import tpu_sc as plsc

N = 10000
E = 320000
D = 128
H = 8
C = 16

NC = 2
NS = 16
L = 16

CHUNK = 80
TILES = NC * NS
EDGES_PER_TILE = E // TILES
NUM_CHUNKS = EDGES_PER_TILE // CHUNK
GROUPS = CHUNK // L
ROWW = 136
ROWS_PER_TILE = N // NS

_INV_SQRT_C = 1.0 / math.sqrt(C)


def _proj_body(x_ref, wq_ref, bq_ref, wk_ref, bk_ref, wv_ref, bv_ref,
               ws_ref, bs_ref, q_ref, kv_ref, skip_ref):
    x = x_ref[...]
    dn = (((1,), (1,)), ((), ()))
    f32 = jnp.float32
    q_ref[...] = lax.dot_general(x, wq_ref[...], dn, preferred_element_type=f32) + bq_ref[...]
    k = lax.dot_general(x, wk_ref[...], dn, preferred_element_type=f32) + bk_ref[...]
    v = lax.dot_general(x, wv_ref[...], dn, preferred_element_type=f32) + bv_ref[...]
    kv_ref[...] = jnp.concatenate([k, v], axis=1)
    skip_ref[...] = x + lax.dot_general(x, ws_ref[...], dn, preferred_element_type=f32) + bs_ref[...]


def _project(x, Wq, bq, Wk, bk, Wv, bv, Wskip, bskip):
    blk = 2000
    grid = (N // blk,)
    full128 = pl.BlockSpec((D, D), lambda i: (0, 0))
    bias = pl.BlockSpec((1, D), lambda i: (0, 0))
    return pl.pallas_call(
        _proj_body,
        grid=grid,
        in_specs=[
            pl.BlockSpec((blk, D), lambda i: (i, 0)),
            full128, bias, full128, bias, full128, bias, full128, bias,
        ],
        out_specs=[
            pl.BlockSpec((blk, D), lambda i: (i, 0)),
            pl.BlockSpec((blk, 2 * D), lambda i: (i, 0)),
            pl.BlockSpec((blk, D), lambda i: (i, 0)),
        ],
        out_shape=[
            jax.ShapeDtypeStruct((N, D), jnp.float32),
            jax.ShapeDtypeStruct((N, 2 * D), jnp.float32),
            jax.ShapeDtypeStruct((N, D), jnp.float32),
        ],
    )(x, Wq, bq.reshape(1, D), Wk, bk.reshape(1, D),
      Wv, bv.reshape(1, D), Wskip, bskip.reshape(1, D))


def _edge_body(q_hbm, kv_hbm, src_hbm, dst_hbm, z_hbm, out_hbm,
               sb0, db0, qd0, kv0, msg0, acc_sh, sem_g0):
    c = lax.axis_index("c")
    s = lax.axis_index("s")

    iota16 = lax.iota(jnp.int32, L)

    base = s * ROWS_PER_TILE
    pltpu.sync_copy(z_hbm, acc_sh.at[pl.ds(base, ROWS_PER_TILE)])
    plsc.subcore_barrier()

    row0 = (c * NS + s) * NUM_CHUNKS

    lane_vecs = [jnp.full((L,), lane, jnp.int32) for lane in range(L)]
    hcols = [h * C + iota16 for h in range(H)]
    last_lane = jnp.full((L,), L - 1, jnp.int32)
    lane0_mask = iota16 == 0
    gdn = lax.GatherDimensionNumbers(offset_dims=(),
                                     collapsed_slice_dims=(0,),
                                     start_index_map=(0,))

    def compute(qd, kvb, msg):
        @plsc.parallel_loop(0, CHUNK, unroll=4)
        def _edge_loop(e):
            ecol = lane_vecs[0] + e
            for h in range(H):
                qh = plsc.load_gather(qd, [ecol, hcols[h]])
                kh = plsc.load_gather(kvb, [ecol, hcols[h]])
                t = plsc.cumsum(qh * kh)
                tot = lax.gather(t, last_lane[:, None], gdn, (1,),
                                 mode=lax.GatherScatterMode.PROMISE_IN_BOUNDS)
                w = jnp.exp(tot * _INV_SQRT_C)
                vh = plsc.load_gather(kvb, [ecol, D + hcols[h]])
                plsc.store_scatter(msg, [ecol, hcols[h]], w * vh)
                plsc.store_scatter(msg, [ecol, lane_vecs[0] + (D + h)],
                                   w, mask=lane0_mask)

    def chunk_body(i, _):
        pltpu.sync_copy(src_hbm.at[row0 + i], sb0)
        pltpu.sync_copy(dst_hbm.at[row0 + i], db0)
        cq = pltpu.async_copy(q_hbm.at[db0], qd0, sem_g0)
        ck = pltpu.async_copy(kv_hbm.at[sb0], kv0, sem_g0)
        cq.wait()
        ck.wait()
        compute(qd0, kv0, msg0)
        pltpu.sync_copy(msg0, acc_sh.at[db0], add=True)
        return 0

    lax.fori_loop(0, NUM_CHUNKS, chunk_body, 0)

    plsc.subcore_barrier()
    out_base = c * N + base
    pltpu.sync_copy(acc_sh.at[pl.ds(base, ROWS_PER_TILE)],
                    out_hbm.at[pl.ds(out_base, ROWS_PER_TILE)])


def _edge_sc(q, kv, src, dst, zeros):
    mesh = plsc.VectorSubcoreMesh(core_axis_name="c", subcore_axis_name="s",
                                  num_cores=NC, num_subcores=NS)
    f = pl.kernel(
        _edge_body,
        out_type=jax.ShapeDtypeStruct((NC * N, ROWW), jnp.float32),
        mesh=mesh,
        scratch_types=[
            pltpu.VMEM((CHUNK,), jnp.int32),
            pltpu.VMEM((CHUNK,), jnp.int32),
            pltpu.VMEM((CHUNK, D), jnp.float32),
            pltpu.VMEM((CHUNK, 2 * D), jnp.float32),
            pltpu.VMEM((CHUNK, ROWW), jnp.float32),
            pltpu.VMEM_SHARED((N, ROWW), jnp.float32),
            pltpu.SemaphoreType.DMA,
        ],
        compiler_params=pltpu.CompilerParams(use_tc_tiling_on_sc=False,
                                             needs_layout_passes=False),
    )
    return f(q, kv, src, dst, zeros)


def _combine_body(acc_ref, skip_ref, p_ref, out_ref):
    a0 = acc_ref[0]
    a1 = acc_ref[1]
    num = a0[:, :D] + a1[:, :D]
    den = a0[:, D:D + H] + a1[:, D:D + H]
    r = 1.0 / (den + 1e-16)
    rexp = lax.dot_general(r, p_ref[...], (((1,), (0,)), ((), ())),
                           preferred_element_type=jnp.float32)
    out_ref[...] = skip_ref[...] + num * rexp


def _combine(acc, skipx, P):
    blk = 2000
    grid = (N // blk,)
    return pl.pallas_call(
        _combine_body,
        grid=grid,
        in_specs=[
            pl.BlockSpec((NC, blk, ROWW), lambda i: (0, i, 0)),
            pl.BlockSpec((blk, D), lambda i: (i, 0)),
            pl.BlockSpec((H, D), lambda i: (0, 0)),
        ],
        out_specs=pl.BlockSpec((blk, D), lambda i: (i, 0)),
        out_shape=jax.ShapeDtypeStruct((N, D), jnp.float32),
    )(acc, skipx, P)


@jax.jit
def kernel(x, edge_index, Wq, bq, Wk, bk, Wv, bv, Wskip, bskip):
    ei = edge_index.astype(jnp.int32)
    src = ei[0].reshape(E // CHUNK, CHUNK)
    dst = ei[1].reshape(E // CHUNK, CHUNK)
    q, kv, skipx = _project(x, Wq, bq, Wk, bk, Wv, bv, Wskip, bskip)
    zeros = jnp.zeros((ROWS_PER_TILE, ROWW), jnp.float32)
    acc_flat = _edge_sc(q, kv, src, dst, zeros)
    acc = acc_flat.reshape(NC, N, ROWW)
    P = jnp.repeat(jnp.eye(H, dtype=jnp.float32), C, axis=1)
    return _combine(acc, skipx, P)


# speedup vs baseline: 3.0741x; 1.0745x over previous
import math

import jax
import jax.numpy as jnp
from jax import lax
from jax.experimental import pallas as pl
from jax.experimental.pallas import tpu as pltpu
from jax.experimental.pallas import tpu_sc as plsc

N = 10000
E = 320000
D = 128
H = 8
C = 16

NC = 2
NS = 16
L = 16

CHUNK = 80
TILES = NC * NS
EDGES_PER_TILE = E // TILES
NUM_CHUNKS = EDGES_PER_TILE // CHUNK
GROUPS = CHUNK // L
ROWW = 136
ROWS_PER_TILE = N // NS

_INV_SQRT_C = 1.0 / math.sqrt(C)


def _proj_body(x_ref, wq_ref, bq_ref, wk_ref, bk_ref, wv_ref, bv_ref,
               ws_ref, bs_ref, q_ref, kv_ref, skip_ref):
    x = x_ref[...]
    dn = (((1,), (1,)), ((), ()))
    f32 = jnp.float32
    q_ref[...] = lax.dot_general(x, wq_ref[...], dn, preferred_element_type=f32) + bq_ref[...]
    k = lax.dot_general(x, wk_ref[...], dn, preferred_element_type=f32) + bk_ref[...]
    v = lax.dot_general(x, wv_ref[...], dn, preferred_element_type=f32) + bv_ref[...]
    kv_ref[...] = jnp.concatenate([k, v], axis=1)
    skip_ref[...] = x + lax.dot_general(x, ws_ref[...], dn, preferred_element_type=f32) + bs_ref[...]


def _project(x, Wq, bq, Wk, bk, Wv, bv, Wskip, bskip):
    blk = 2000
    grid = (N // blk,)
    full128 = pl.BlockSpec((D, D), lambda i: (0, 0))
    bias = pl.BlockSpec((1, D), lambda i: (0, 0))
    return pl.pallas_call(
        _proj_body,
        grid=grid,
        in_specs=[
            pl.BlockSpec((blk, D), lambda i: (i, 0)),
            full128, bias, full128, bias, full128, bias, full128, bias,
        ],
        out_specs=[
            pl.BlockSpec((blk, D), lambda i: (i, 0)),
            pl.BlockSpec((blk, 2 * D), lambda i: (i, 0)),
            pl.BlockSpec((blk, D), lambda i: (i, 0)),
        ],
        out_shape=[
            jax.ShapeDtypeStruct((N, D), jnp.float32),
            jax.ShapeDtypeStruct((N, 2 * D), jnp.float32),
            jax.ShapeDtypeStruct((N, D), jnp.float32),
        ],
    )(x, Wq, bq.reshape(1, D), Wk, bk.reshape(1, D),
      Wv, bv.reshape(1, D), Wskip, bskip.reshape(1, D))


def _edge_body(q_hbm, kv_hbm, ei_hbm, z_hbm, out_hbm,
               sd0, qd0, kv0, msg0, acc_sh, sem_g0):
    c = lax.axis_index("c")
    s = lax.axis_index("s")

    iota16 = lax.iota(jnp.int32, L)

    base = s * ROWS_PER_TILE
    pltpu.sync_copy(z_hbm, acc_sh.at[pl.ds(base, ROWS_PER_TILE)])
    plsc.subcore_barrier()

    row0 = (c * NS + s) * NUM_CHUNKS

    lane_vecs = [jnp.full((L,), lane, jnp.int32) for lane in range(L)]
    hcols = [h * C + iota16 for h in range(H)]
    last_lane = jnp.full((L,), L - 1, jnp.int32)
    lane0_mask = iota16 == 0
    gdn = lax.GatherDimensionNumbers(offset_dims=(),
                                     collapsed_slice_dims=(0,),
                                     start_index_map=(0,))

    def compute(qd, kvb, msg):
        @plsc.parallel_loop(0, CHUNK, unroll=4)
        def _edge_loop(e):
            ecol = lane_vecs[0] + e
            for h in range(H):
                qh = plsc.load_gather(qd, [ecol, hcols[h]])
                kh = plsc.load_gather(kvb, [ecol, hcols[h]])
                t = plsc.cumsum(qh * kh)
                tot = lax.gather(t, last_lane[:, None], gdn, (1,),
                                 mode=lax.GatherScatterMode.PROMISE_IN_BOUNDS)
                w = jnp.exp(tot * _INV_SQRT_C)
                vh = plsc.load_gather(kvb, [ecol, D + hcols[h]])
                plsc.store_scatter(msg, [ecol, hcols[h]], w * vh)
                plsc.store_scatter(msg, [ecol, lane_vecs[0] + (D + h)],
                                   w, mask=lane0_mask)

    def chunk_body(i, _):
        pltpu.sync_copy(ei_hbm.at[:, row0 + i], sd0)
        cq = pltpu.async_copy(q_hbm.at[sd0.at[1]], qd0, sem_g0)
        ck = pltpu.async_copy(kv_hbm.at[sd0.at[0]], kv0, sem_g0)
        cq.wait()
        ck.wait()
        compute(qd0, kv0, msg0)
        pltpu.sync_copy(msg0, acc_sh.at[sd0.at[1]], add=True)
        return 0

    lax.fori_loop(0, NUM_CHUNKS, chunk_body, 0)

    plsc.subcore_barrier()
    out_base = c * N + base
    pltpu.sync_copy(acc_sh.at[pl.ds(base, ROWS_PER_TILE)],
                    out_hbm.at[pl.ds(out_base, ROWS_PER_TILE)])


def _edge_sc(q, kv, ei3, zeros):
    mesh = plsc.VectorSubcoreMesh(core_axis_name="c", subcore_axis_name="s",
                                  num_cores=NC, num_subcores=NS)
    f = pl.kernel(
        _edge_body,
        out_type=jax.ShapeDtypeStruct((NC * N, ROWW), jnp.float32),
        mesh=mesh,
        scratch_types=[
            pltpu.VMEM((2, CHUNK), jnp.int32),
            pltpu.VMEM((CHUNK, D), jnp.float32),
            pltpu.VMEM((CHUNK, 2 * D), jnp.float32),
            pltpu.VMEM((CHUNK, ROWW), jnp.float32),
            pltpu.VMEM_SHARED((N, ROWW), jnp.float32),
            pltpu.SemaphoreType.DMA,
        ],
        compiler_params=pltpu.CompilerParams(use_tc_tiling_on_sc=False,
                                             needs_layout_passes=False),
    )
    return f(q, kv, ei3, zeros)


def _combine_body(acc_ref, skip_ref, p_ref, out_ref):
    a0 = acc_ref[0]
    a1 = acc_ref[1]
    num = a0[:, :D] + a1[:, :D]
    den = a0[:, D:D + H] + a1[:, D:D + H]
    r = 1.0 / (den + 1e-16)
    rexp = lax.dot_general(r, p_ref[...], (((1,), (0,)), ((), ())),
                           preferred_element_type=jnp.float32)
    out_ref[...] = skip_ref[...] + num * rexp


def _combine(acc, skipx, P):
    blk = 2000
    grid = (N // blk,)
    return pl.pallas_call(
        _combine_body,
        grid=grid,
        in_specs=[
            pl.BlockSpec((NC, blk, ROWW), lambda i: (0, i, 0)),
            pl.BlockSpec((blk, D), lambda i: (i, 0)),
            pl.BlockSpec((H, D), lambda i: (0, 0)),
        ],
        out_specs=pl.BlockSpec((blk, D), lambda i: (i, 0)),
        out_shape=jax.ShapeDtypeStruct((N, D), jnp.float32),
    )(acc, skipx, P)


@jax.jit
def kernel(x, edge_index, Wq, bq, Wk, bk, Wv, bv, Wskip, bskip):
    ei3 = edge_index.astype(jnp.int32).reshape(2, E // CHUNK, CHUNK)
    q, kv, skipx = _project(x, Wq, bq, Wk, bk, Wv, bv, Wskip, bskip)
    zeros = jnp.zeros((ROWS_PER_TILE, ROWW), jnp.float32)
    acc_flat = _edge_sc(q, kv, ei3, zeros)
    acc = acc_flat.reshape(NC, N, ROWW)
    P = jnp.repeat(jnp.eye(H, dtype=jnp.float32), C, axis=1)
    return _combine(acc, skipx, P)
